# initial kernel scaffold (unmeasured)
import jax
import jax.numpy as jnp
from jax import lax
from jax.experimental import pallas as pl
from jax.experimental.pallas import tpu as pltpu

B = 8
H = 8
D = 128
BS = 16
P_LOCAL = 512
KL = P_LOCAL * BS
NSLOTS = 512


def kernel(Q, K, V, bt, lens):
    lens2d = lens.reshape(B, 1)

    def body(q_ref, k_ref, v_ref, bt_ref, lens_ref, out_ref,
             acc_buf, stat_buf, send_sems, recv_sems):
        my_x = lax.axis_index("x")
        my_y = lax.axis_index("y")
        peer = (my_x, 1 - my_y)

        barrier = pltpu.get_barrier_semaphore()
        pl.semaphore_signal(barrier, inc=1, device_id=peer,
                            device_id_type=pl.DeviceIdType.MESH)
        pl.semaphore_wait(barrier, 1)

        offset = my_y * P_LOCAL
        counts_rows = []
        for b in range(B):
            bt_row = bt_ref[b:b + 1, :]
            pg = lax.broadcasted_iota(jnp.int32, (P_LOCAL, NSLOTS), 0) + offset
            sl = lax.broadcasted_iota(jnp.int32, (P_LOCAL, NSLOTS), 1)
            len_b = lens_ref[b, 0]
            match = (bt_row == pg) & (sl < len_b)
            cnt = jnp.sum(match.astype(jnp.float32), axis=1)
            ck = jnp.broadcast_to(cnt[:, None], (P_LOCAL, BS)).reshape(KL)
            counts_rows.append(ck)
        counts = jnp.stack(counts_rows, axis=0)

        q = q_ref[:, 0, :, :]
        k = k_ref[...].reshape(KL, H, D)
        v = v_ref[...].reshape(KL, H, D)

        s = lax.dot_general(
            q, k,
            dimension_numbers=(((2,), (2,)), ((1,), (1,))),
            preferred_element_type=jnp.float32,
        ) * (D ** -0.5)

        m = jnp.max(s, axis=2)
        p = jnp.exp(s - m[:, :, None]) * counts[None, :, :]
        l = jnp.sum(p, axis=2)
        acc = lax.dot_general(
            p, v,
            dimension_numbers=(((2,), (0,)), ((0,), (1,))),
            preferred_element_type=jnp.float32,
        )

        acc_buf[0] = acc
        stat_buf[0, 0] = m
        stat_buf[0, 1] = l

        rdma_acc = pltpu.make_async_remote_copy(
            src_ref=acc_buf.at[0], dst_ref=acc_buf.at[1],
            send_sem=send_sems.at[0], recv_sem=recv_sems.at[0],
            device_id=peer, device_id_type=pl.DeviceIdType.MESH,
        )
        rdma_stat = pltpu.make_async_remote_copy(
            src_ref=stat_buf.at[0], dst_ref=stat_buf.at[1],
            send_sem=send_sems.at[1], recv_sem=recv_sems.at[1],
            device_id=peer, device_id_type=pl.DeviceIdType.MESH,
        )
        rdma_acc.start()
        rdma_stat.start()
        rdma_acc.wait()
        rdma_stat.wait()

        m_mine = stat_buf[0, 0]
        l_mine = stat_buf[0, 1]
        m_peer = stat_buf[1, 0]
        l_peer = stat_buf[1, 1]
        m_g = jnp.maximum(m_mine, m_peer)
        a_mine = jnp.exp(m_mine - m_g)
        a_peer = jnp.exp(m_peer - m_g)
        l_g = l_mine * a_mine + l_peer * a_peer
        o = (acc_buf[0] * a_mine[:, :, None]
             + acc_buf[1] * a_peer[:, :, None]) / l_g[:, :, None]

        out_ref[...] = o.transpose(1, 0, 2).reshape(B, 1, H, D)

    return pl.pallas_call(
        body,
        out_shape=jax.ShapeDtypeStruct((B, 1, H, D), jnp.float32),
        in_specs=[
            pl.BlockSpec(memory_space=pltpu.VMEM),
            pl.BlockSpec(memory_space=pltpu.VMEM),
            pl.BlockSpec(memory_space=pltpu.VMEM),
            pl.BlockSpec(memory_space=pltpu.VMEM),
            pl.BlockSpec(memory_space=pltpu.SMEM),
        ],
        out_specs=pl.BlockSpec(memory_space=pltpu.VMEM),
        scratch_shapes=[
            pltpu.VMEM((2, H, B, D), jnp.float32),
            pltpu.VMEM((2, 2, H, B), jnp.float32),
            pltpu.SemaphoreType.DMA((2,)),
            pltpu.SemaphoreType.DMA((2,)),
        ],
        compiler_params=pltpu.CompilerParams(collective_id=0),
    )(Q, K, V, bt, lens2d)


# baseline (device time: 47688 ns/iter reference)
import jax
import jax.numpy as jnp
from jax import lax
from jax.experimental import pallas as pl
from jax.experimental.pallas import tpu as pltpu

B = 8
H = 8
D = 128
BS = 16
P_LOCAL = 512
NSLOTS = 512
CP = 64
CKL = CP * BS
N_CHUNKS = P_LOCAL // CP
NEG_INF = -1e30


def kernel(Q, K, V, bt, lens):
    lens2d = lens.reshape(B, 1)

    def body(q_ref, k_ref, v_ref, bt_ref, lens_ref, out_ref,
             acc_s, m_s, l_s, acc_buf, stat_buf, send_sems, recv_sems):
        c = pl.program_id(0)
        my_x = lax.axis_index("x")
        my_y = lax.axis_index("y")
        peer = (my_x, 1 - my_y)

        @pl.when(c == 0)
        def _init():
            acc_s[...] = jnp.zeros((H, B, D), jnp.float32)
            m_s[...] = jnp.full((H, B), NEG_INF, jnp.float32)
            l_s[...] = jnp.zeros((H, B), jnp.float32)

        base = my_y * P_LOCAL + c * CP
        counts_rows = []
        for b in range(B):
            bt_row = bt_ref[b:b + 1, :]
            pg = lax.broadcasted_iota(jnp.int32, (CP, NSLOTS), 0) + base
            sl = lax.broadcasted_iota(jnp.int32, (CP, NSLOTS), 1)
            len_b = lens_ref[b, 0]
            match = (bt_row == pg) & (sl < len_b)
            cnt = jnp.sum(match.astype(jnp.float32), axis=1, keepdims=True)
            counts_rows.append(cnt.T)
        counts_pg = jnp.concatenate(counts_rows, axis=0)

        row = lax.broadcasted_iota(jnp.int32, (CP, CKL), 0)
        col = lax.broadcasted_iota(jnp.int32, (CP, CKL), 1)
        expand = (col // BS == row).astype(jnp.float32)
        counts = lax.dot_general(
            counts_pg, expand,
            dimension_numbers=(((1,), (0,)), ((), ())),
            preferred_element_type=jnp.float32,
        )

        q = q_ref[:, 0, :, :]
        k = k_ref[...].reshape(CKL, H, D)
        v = v_ref[...].reshape(CKL, H, D)

        s_heads = []
        for h in range(H):
            s_h = lax.dot_general(
                q[:, h, :], k[:, h, :],
                dimension_numbers=(((1,), (1,)), ((), ())),
                preferred_element_type=jnp.float32,
            )
            s_heads.append(s_h[None])
        s = jnp.concatenate(s_heads, axis=0) * (D ** -0.5)

        m_old = m_s[...]
        cm = jnp.max(s, axis=2)
        m_new = jnp.maximum(m_old, cm)
        alpha = jnp.exp(m_old - m_new)
        p = jnp.exp(s - m_new[:, :, None]) * counts[None]
        l_s[...] = l_s[...] * alpha + jnp.sum(p, axis=2)
        pv_heads = []
        for h in range(H):
            pv_h = lax.dot_general(
                p[h], v[:, h, :],
                dimension_numbers=(((1,), (0,)), ((), ())),
                preferred_element_type=jnp.float32,
            )
            pv_heads.append(pv_h[None])
        pv = jnp.concatenate(pv_heads, axis=0)
        acc_s[...] = acc_s[...] * alpha[:, :, None] + pv
        m_s[...] = m_new

        @pl.when(c == N_CHUNKS - 1)
        def _exchange_and_combine():
            barrier = pltpu.get_barrier_semaphore()
            pl.semaphore_signal(barrier, inc=1, device_id=peer,
                                device_id_type=pl.DeviceIdType.MESH)
            pl.semaphore_wait(barrier, 1)

            acc_buf[0] = acc_s[...]
            stat_buf[0, 0] = m_s[...]
            stat_buf[0, 1] = l_s[...]

            rdma_acc = pltpu.make_async_remote_copy(
                src_ref=acc_buf.at[0], dst_ref=acc_buf.at[1],
                send_sem=send_sems.at[0], recv_sem=recv_sems.at[0],
                device_id=peer, device_id_type=pl.DeviceIdType.MESH,
            )
            rdma_stat = pltpu.make_async_remote_copy(
                src_ref=stat_buf.at[0], dst_ref=stat_buf.at[1],
                send_sem=send_sems.at[1], recv_sem=recv_sems.at[1],
                device_id=peer, device_id_type=pl.DeviceIdType.MESH,
            )
            rdma_acc.start()
            rdma_stat.start()
            rdma_acc.wait()
            rdma_stat.wait()

            m_mine = stat_buf[0, 0]
            l_mine = stat_buf[0, 1]
            m_peer = stat_buf[1, 0]
            l_peer = stat_buf[1, 1]
            m_g = jnp.maximum(m_mine, m_peer)
            a_mine = jnp.exp(m_mine - m_g)
            a_peer = jnp.exp(m_peer - m_g)
            l_g = l_mine * a_mine + l_peer * a_peer
            o = (acc_buf[0] * a_mine[:, :, None]
                 + acc_buf[1] * a_peer[:, :, None]) / l_g[:, :, None]
            out_ref[...] = o.transpose(1, 0, 2).reshape(B, 1, H, D)

    return pl.pallas_call(
        body,
        grid=(N_CHUNKS,),
        out_shape=jax.ShapeDtypeStruct((B, 1, H, D), jnp.float32),
        in_specs=[
            pl.BlockSpec((B, 1, H, D), lambda c: (0, 0, 0, 0),
                         memory_space=pltpu.VMEM),
            pl.BlockSpec((CP, BS, H, D), lambda c: (c, 0, 0, 0),
                         memory_space=pltpu.VMEM),
            pl.BlockSpec((CP, BS, H, D), lambda c: (c, 0, 0, 0),
                         memory_space=pltpu.VMEM),
            pl.BlockSpec((B, NSLOTS), lambda c: (0, 0),
                         memory_space=pltpu.VMEM),
            pl.BlockSpec((B, 1), lambda c: (0, 0),
                         memory_space=pltpu.SMEM),
        ],
        out_specs=pl.BlockSpec((B, 1, H, D), lambda c: (0, 0, 0, 0),
                               memory_space=pltpu.VMEM),
        scratch_shapes=[
            pltpu.VMEM((H, B, D), jnp.float32),
            pltpu.VMEM((H, B), jnp.float32),
            pltpu.VMEM((H, B), jnp.float32),
            pltpu.VMEM((2, H, B, D), jnp.float32),
            pltpu.VMEM((2, 2, H, B), jnp.float32),
            pltpu.SemaphoreType.DMA((2,)),
            pltpu.SemaphoreType.DMA((2,)),
        ],
        compiler_params=pltpu.CompilerParams(
            collective_id=0,
            dimension_semantics=("arbitrary",),
        ),
    )(Q, K, V, bt, lens2d)


# device time: 32747 ns/iter; 1.4563x vs baseline; 1.4563x over previous
import jax
import jax.numpy as jnp
from jax import lax
from jax.experimental import pallas as pl
from jax.experimental.pallas import tpu as pltpu

B = 8
H = 8
D = 128
BS = 16
P_LOCAL = 512
P_X = P_LOCAL // 2
NSLOTS = 512
CP = 64
CKL = CP * BS
N_CHUNKS = P_X // CP
NEG_INF = -1e30


def kernel(Q, K, V, bt, lens):
    xy = jnp.stack([lax.axis_index("x"), lax.axis_index("y")])

    def body(xy_ref, lens_ref, q_ref, k_ref, v_ref, bt_ref, out_ref,
             acc_s, m_s, l_s, acc_buf, stat_buf, send_sems, recv_sems):
        c = pl.program_id(0)
        my_x = xy_ref[0]
        my_y = xy_ref[1]
        peer_x = (1 - my_x, my_y)
        peer_y = (my_x, 1 - my_y)

        @pl.when(c == 0)
        def _init():
            acc_s[...] = jnp.zeros((H, B, D), jnp.float32)
            m_s[...] = jnp.full((H, B), NEG_INF, jnp.float32)
            l_s[...] = jnp.zeros((H, B), jnp.float32)

        base = my_y * P_LOCAL + my_x * P_X + c * CP
        counts_rows = []
        for b in range(B):
            bt_row = bt_ref[b:b + 1, :]
            pg = lax.broadcasted_iota(jnp.int32, (CP, NSLOTS), 0) + base
            sl = lax.broadcasted_iota(jnp.int32, (CP, NSLOTS), 1)
            len_b = lens_ref[b]
            match = (bt_row == pg) & (sl < len_b)
            cnt = jnp.sum(match.astype(jnp.float32), axis=1, keepdims=True)
            counts_rows.append(cnt.T)
        counts_pg = jnp.concatenate(counts_rows, axis=0)

        row = lax.broadcasted_iota(jnp.int32, (CP, CKL), 0)
        col = lax.broadcasted_iota(jnp.int32, (CP, CKL), 1)
        expand = (col // BS == row).astype(jnp.float32)
        counts = lax.dot_general(
            counts_pg, expand,
            dimension_numbers=(((1,), (0,)), ((), ())),
            preferred_element_type=jnp.float32,
        )

        q = q_ref[:, 0, :, :]
        k = k_ref[...].reshape(CKL, H, D)
        v = v_ref[...].reshape(CKL, H, D)

        s_heads = []
        for h in range(H):
            s_h = lax.dot_general(
                q[:, h, :], k[:, h, :],
                dimension_numbers=(((1,), (1,)), ((), ())),
                preferred_element_type=jnp.float32,
            )
            s_heads.append(s_h[None])
        s = jnp.concatenate(s_heads, axis=0) * (D ** -0.5)

        m_old = m_s[...]
        cm = jnp.max(s, axis=2)
        m_new = jnp.maximum(m_old, cm)
        alpha = jnp.exp(m_old - m_new)
        p = jnp.exp(s - m_new[:, :, None]) * counts[None]
        l_s[...] = l_s[...] * alpha + jnp.sum(p, axis=2)
        pv_heads = []
        for h in range(H):
            pv_h = lax.dot_general(
                p[h], v[:, h, :],
                dimension_numbers=(((1,), (0,)), ((), ())),
                preferred_element_type=jnp.float32,
            )
            pv_heads.append(pv_h[None])
        pv = jnp.concatenate(pv_heads, axis=0)
        acc_s[...] = acc_s[...] * alpha[:, :, None] + pv
        m_s[...] = m_new

        @pl.when(c == N_CHUNKS - 1)
        def _exchange_and_combine():
            barrier = pltpu.get_barrier_semaphore()
            for nbr in (peer_x, peer_y):
                pl.semaphore_signal(barrier, inc=1, device_id=nbr,
                                    device_id_type=pl.DeviceIdType.MESH)
            pl.semaphore_wait(barrier, 2)

            def exchange(peer, recv_slot, sem_base):
                rdma_acc = pltpu.make_async_remote_copy(
                    src_ref=acc_buf.at[0], dst_ref=acc_buf.at[recv_slot],
                    send_sem=send_sems.at[sem_base], recv_sem=recv_sems.at[sem_base],
                    device_id=peer, device_id_type=pl.DeviceIdType.MESH,
                )
                rdma_stat = pltpu.make_async_remote_copy(
                    src_ref=stat_buf.at[0], dst_ref=stat_buf.at[recv_slot],
                    send_sem=send_sems.at[sem_base + 1],
                    recv_sem=recv_sems.at[sem_base + 1],
                    device_id=peer, device_id_type=pl.DeviceIdType.MESH,
                )
                rdma_acc.start()
                rdma_stat.start()
                rdma_acc.wait()
                rdma_stat.wait()

            def combine(slot):
                m_a = stat_buf[0, 0]
                l_a = stat_buf[0, 1]
                m_b = stat_buf[slot, 0]
                l_b = stat_buf[slot, 1]
                m_g = jnp.maximum(m_a, m_b)
                w_a = jnp.exp(m_a - m_g)
                w_b = jnp.exp(m_b - m_g)
                l_g = l_a * w_a + l_b * w_b
                acc_g = (acc_buf[0] * w_a[:, :, None]
                         + acc_buf[slot] * w_b[:, :, None])
                return m_g, l_g, acc_g

            acc_buf[0] = acc_s[...]
            stat_buf[0, 0] = m_s[...]
            stat_buf[0, 1] = l_s[...]

            exchange(peer_x, 1, 0)
            m_g, l_g, acc_g = combine(1)
            acc_buf[0] = acc_g
            stat_buf[0, 0] = m_g
            stat_buf[0, 1] = l_g

            exchange(peer_y, 2, 2)
            m_g, l_g, acc_g = combine(2)

            o = acc_g / l_g[:, :, None]
            out_ref[...] = o.transpose(1, 0, 2).reshape(B, 1, H, D)

    grid_spec = pltpu.PrefetchScalarGridSpec(
        num_scalar_prefetch=2,
        grid=(N_CHUNKS,),
        in_specs=[
            pl.BlockSpec((B, 1, H, D), lambda c, xy, lens: (0, 0, 0, 0)),
            pl.BlockSpec((CP, BS, H, D),
                         lambda c, xy, lens: (xy[0] * N_CHUNKS + c, 0, 0, 0)),
            pl.BlockSpec((CP, BS, H, D),
                         lambda c, xy, lens: (xy[0] * N_CHUNKS + c, 0, 0, 0)),
            pl.BlockSpec((B, NSLOTS), lambda c, xy, lens: (0, 0)),
        ],
        out_specs=pl.BlockSpec((B, 1, H, D), lambda c, xy, lens: (0, 0, 0, 0)),
        scratch_shapes=[
            pltpu.VMEM((H, B, D), jnp.float32),
            pltpu.VMEM((H, B), jnp.float32),
            pltpu.VMEM((H, B), jnp.float32),
            pltpu.VMEM((3, H, B, D), jnp.float32),
            pltpu.VMEM((3, 2, H, B), jnp.float32),
            pltpu.SemaphoreType.DMA((4,)),
            pltpu.SemaphoreType.DMA((4,)),
        ],
    )
    return pl.pallas_call(
        body,
        grid_spec=grid_spec,
        out_shape=jax.ShapeDtypeStruct((B, 1, H, D), jnp.float32),
        compiler_params=pltpu.CompilerParams(
            collective_id=0,
            dimension_semantics=("arbitrary",),
        ),
    )(xy, lens, Q, K, V, bt)
